# Initial kernel scaffold; baseline (speedup 1.0000x reference)
import jax
import jax.numpy as jnp
from jax.experimental import pallas as pl


def kernel(x, edge_index, edge_weight, W0l, b0l, W0r, W1l, b1l, W1r, Wskip, a0, a1):
    raise NotImplementedError("write your pallas kernel here")


# speedup vs baseline: 5.0545x; 5.0545x over previous
import functools

import jax
import jax.numpy as jnp
from jax import lax
from jax.experimental import pallas as pl
from jax.experimental.pallas import tpu as pltpu
from jax.experimental.pallas import tpu_sc as plsc

N = 10000
E = 320000
D = 128

NC = 2
NS = 16
NW = NC * NS
EPT = E // NW
CH = 80
NCHUNK = EPT // CH
N_PAD = 10240
ROWS_PT = N_PAD // NS
WORDS_PT = N_PAD // NS


def _seg_sum_body(with_deg, *refs):
    if with_deg:
        (h_hbm, src_hbm, dst_hbm, z128_hbm, z1d_hbm,
         out_hbm, deg_hbm,
         srcv, dstv, rows, hist, cbuf, res, acc, stag, sem) = refs
    else:
        (h_hbm, src_hbm, dst_hbm, z128_hbm,
         out_hbm,
         srcv, dstv, rows, acc, sem) = refs

    cid = lax.axis_index("c")
    sid = lax.axis_index("s")
    wid = sid * NC + cid

    rbase = sid * ROWS_PT
    pltpu.sync_copy(z128_hbm, acc.at[pl.ds(rbase, ROWS_PT)])
    if with_deg:
        pltpu.sync_copy(z1d_hbm, hist)
    plsc.subcore_barrier()

    ebase = wid * EPT
    ones16 = jnp.ones((16,), jnp.float32)

    def step(i, carry):
        off = ebase + i * CH
        pltpu.sync_copy(src_hbm.at[pl.ds(off, CH)], srcv)
        pltpu.sync_copy(dst_hbm.at[pl.ds(off, CH)], dstv)
        pltpu.async_copy(h_hbm.at[srcv], rows, sem).wait()
        pltpu.sync_copy(rows, acc.at[dstv], add=True)
        if with_deg:
            for j in range(CH // 16):
                d16 = dstv[pl.ds(j * 16, 16)]
                plsc.addupdate_scatter(hist, [d16], ones16)
        return carry

    lax.fori_loop(0, NCHUNK, step, 0)

    if with_deg:
        pltpu.sync_copy(hist, stag.at[sid])
    plsc.subcore_barrier()

    pltpu.sync_copy(acc.at[pl.ds(rbase, ROWS_PT)],
                    out_hbm.at[cid, pl.ds(rbase, ROWS_PT)])
    if with_deg:
        wbase = sid * WORDS_PT
        for t in range(NS):
            pltpu.sync_copy(stag.at[t, pl.ds(wbase, WORDS_PT)], cbuf.at[t])
        for j in range(WORDS_PT // 16):
            tot = cbuf[0, pl.ds(j * 16, 16)]
            for t in range(1, NS):
                tot = tot + cbuf[t, pl.ds(j * 16, 16)]
            res[pl.ds(j * 16, 16)] = tot
        pltpu.sync_copy(res, deg_hbm.at[cid, pl.ds(wbase, WORDS_PT)])


def _make_seg_sum(with_deg):
    mesh = plsc.VectorSubcoreMesh(core_axis_name="c", subcore_axis_name="s",
                                  num_cores=NC, num_subcores=NS)
    out_type = [jax.ShapeDtypeStruct((NC, N_PAD, D), jnp.float32)]
    scratch = [
        pltpu.VMEM((CH,), jnp.int32),
        pltpu.VMEM((CH,), jnp.int32),
        pltpu.VMEM((CH, D), jnp.float32),
    ]
    if with_deg:
        out_type.append(jax.ShapeDtypeStruct((NC, N_PAD), jnp.float32))
        scratch.append(pltpu.VMEM((N_PAD,), jnp.float32))
        scratch.append(pltpu.VMEM((NS, WORDS_PT), jnp.float32))
        scratch.append(pltpu.VMEM((WORDS_PT,), jnp.float32))
        scratch.append(pltpu.VMEM_SHARED((N_PAD, D), jnp.float32))
        scratch.append(pltpu.VMEM_SHARED((NS, N_PAD), jnp.float32))
    else:
        scratch.append(pltpu.VMEM_SHARED((N_PAD, D), jnp.float32))
    scratch.append(pltpu.SemaphoreType.DMA)

    return pl.kernel(
        functools.partial(_seg_sum_body, with_deg),
        out_type=tuple(out_type) if with_deg else out_type[0],
        mesh=mesh,
        scratch_types=scratch,
        compiler_params=pltpu.CompilerParams(needs_layout_passes=False),
    )


_seg_sum_deg = _make_seg_sum(True)
_seg_sum = _make_seg_sum(False)


def _prelu(v, a):
    return jnp.where(v >= 0, v, a * v)


RB = 1000


def _layer1_body(agg_ref, deg0_ref, deg1_ref, x_ref, w0l_ref, b0l_ref,
                 w0r_ref, wskip_ref, a0_ref, out_ref):
    agg = agg_ref[0] + agg_ref[1]
    deg = deg0_ref[...] + deg1_ref[...]
    mean = agg / jnp.maximum(deg, 1.0)
    xb = x_ref[...]
    a0 = a0_ref[...]
    h = (jnp.dot(mean, w0l_ref[...], preferred_element_type=jnp.float32)
         + b0l_ref[...]
         + jnp.dot(xb, w0r_ref[...], preferred_element_type=jnp.float32))
    h = _prelu(_prelu(h, a0), a0)
    out_ref[...] = h + jnp.dot(xb, wskip_ref[...],
                               preferred_element_type=jnp.float32)


def _layer2_body(agg_ref, deg0_ref, deg1_ref, h_ref, w1l_ref, b1l_ref,
                 w1r_ref, a1_ref, out_ref):
    agg = agg_ref[0] + agg_ref[1]
    deg = deg0_ref[...] + deg1_ref[...]
    mean = agg / jnp.maximum(deg, 1.0)
    h = (jnp.dot(mean, w1l_ref[...], preferred_element_type=jnp.float32)
         + b1l_ref[...]
         + jnp.dot(h_ref[...], w1r_ref[...], preferred_element_type=jnp.float32))
    out_ref[...] = _prelu(h, a1_ref[...])


_full128 = pl.BlockSpec((D, D), lambda i: (0, 0))
_row1 = pl.BlockSpec((1, D), lambda i: (0, 0))


_degspec = pl.BlockSpec((RB, 1), lambda i: (i, 0))


def _tc_layer1(aggp, deg0, deg1, x, W0l, b0l, W0r, Wskip, a0):
    return pl.pallas_call(
        _layer1_body,
        grid=(N // RB,),
        in_specs=[
            pl.BlockSpec((NC, RB, D), lambda i: (0, i, 0)),
            _degspec, _degspec,
            pl.BlockSpec((RB, D), lambda i: (i, 0)),
            _full128, _row1, _full128, _full128, _row1,
        ],
        out_specs=pl.BlockSpec((RB, D), lambda i: (i, 0)),
        out_shape=jax.ShapeDtypeStruct((N, D), jnp.float32),
    )(aggp, deg0, deg1, x, W0l, b0l, W0r, Wskip, a0)


def _tc_layer2(aggp, deg0, deg1, h, W1l, b1l, W1r, a1):
    return pl.pallas_call(
        _layer2_body,
        grid=(N // RB,),
        in_specs=[
            pl.BlockSpec((NC, RB, D), lambda i: (0, i, 0)),
            _degspec, _degspec,
            pl.BlockSpec((RB, D), lambda i: (i, 0)),
            _full128, _row1, _full128, _row1,
        ],
        out_specs=pl.BlockSpec((RB, D), lambda i: (i, 0)),
        out_shape=jax.ShapeDtypeStruct((N, D), jnp.float32),
    )(aggp, deg0, deg1, h, W1l, b1l, W1r, a1)


def kernel(x, edge_index, edge_weight, W0l, b0l, W0r, W1l, b1l, W1r,
           Wskip, a0, a1):
    del edge_weight
    src = jnp.asarray(edge_index[0], jnp.int32)
    dst = jnp.asarray(edge_index[1], jnp.int32)
    z128 = jnp.zeros((ROWS_PT, D), jnp.float32)
    b0l2 = b0l.reshape(1, D)
    b1l2 = b1l.reshape(1, D)
    a02 = a0.reshape(1, D)
    a12 = a1.reshape(1, D)

    z1d = jnp.zeros((N_PAD,), jnp.float32)
    agg1p, degp = _seg_sum_deg(x, src, dst, z128, z1d)
    deg0 = degp[0].reshape(N_PAD, 1)[:N]
    deg1 = degp[1].reshape(N_PAD, 1)[:N]
    h2in = _tc_layer1(agg1p, deg0, deg1, x, W0l, b0l2, W0r, Wskip, a02)
    agg2p = _seg_sum(h2in, src, dst, z128)
    return _tc_layer2(agg2p, deg0, deg1, h2in, W1l, b1l2, W1r, a12)


# speedup vs baseline: 7.9100x; 1.5649x over previous
import functools

import jax
import jax.numpy as jnp
from jax import lax
from jax.experimental import pallas as pl
from jax.experimental.pallas import tpu as pltpu
from jax.experimental.pallas import tpu_sc as plsc

N = 10000
E = 320000
D = 128

NC = 2
NS = 16
NW = NC * NS
EPT = E // NW
CH = 80
NCHUNK = EPT // CH
N_PAD = 10240
ROWS_PT = N_PAD // NS
WORDS_PT = N_PAD // NS


def _seg_sum_body(with_deg, *refs):
    if with_deg:
        (h_hbm, src_hbm, dst_hbm, z128_hbm, z1d_hbm,
         out_hbm, deg_hbm,
         srcv0, srcv1, dstv0, dstv1, rows0, rows1,
         hist, cbuf, res, acc, stag, sem0, sem1) = refs
    else:
        (h_hbm, src_hbm, dst_hbm, z128_hbm,
         out_hbm,
         srcv0, srcv1, dstv0, dstv1, rows0, rows1,
         acc, sem0, sem1) = refs

    cid = lax.axis_index("c")
    sid = lax.axis_index("s")
    wid = sid * NC + cid

    rbase = sid * ROWS_PT
    pltpu.sync_copy(z128_hbm, acc.at[pl.ds(rbase, ROWS_PT)])
    if with_deg:
        pltpu.sync_copy(z1d_hbm, hist)
    plsc.subcore_barrier()

    ebase = wid * EPT
    ones16 = jnp.ones((16,), jnp.float32)

    def load_idx(off, srcv, dstv):
        pltpu.sync_copy(src_hbm.at[pl.ds(off, CH)], srcv)
        pltpu.sync_copy(dst_hbm.at[pl.ds(off, CH)], dstv)

    def consume(rows, srcv, dstv, sem):
        pltpu.make_async_copy(h_hbm.at[srcv], rows, sem).wait()
        pltpu.sync_copy(rows, acc.at[dstv], add=True)
        if with_deg:
            for j in range(CH // 16):
                d16 = dstv[pl.ds(j * 16, 16)]
                plsc.addupdate_scatter(hist, [d16], ones16)

    load_idx(ebase, srcv0, dstv0)
    pltpu.async_copy(h_hbm.at[srcv0], rows0, sem0)

    def step(k, carry):
        o1 = ebase + (2 * k + 1) * CH
        load_idx(o1, srcv1, dstv1)
        pltpu.async_copy(h_hbm.at[srcv1], rows1, sem1)
        consume(rows0, srcv0, dstv0, sem0)
        o2 = ebase + (2 * k + 2) * CH
        load_idx(o2, srcv0, dstv0)
        pltpu.async_copy(h_hbm.at[srcv0], rows0, sem0)
        consume(rows1, srcv1, dstv1, sem1)
        return carry

    lax.fori_loop(0, NCHUNK // 2, step, 0)
    consume(rows0, srcv0, dstv0, sem0)

    if with_deg:
        pltpu.sync_copy(hist, stag.at[sid])
    plsc.subcore_barrier()

    pltpu.sync_copy(acc.at[pl.ds(rbase, ROWS_PT)],
                    out_hbm.at[cid, pl.ds(rbase, ROWS_PT)])
    if with_deg:
        wbase = sid * WORDS_PT
        pltpu.sync_copy(stag.at[0, pl.ds(wbase, WORDS_PT)], res)
        for t in range(1, NS):
            pltpu.sync_copy(stag.at[t, pl.ds(wbase, WORDS_PT)], cbuf)
            for j in range(WORDS_PT // 16):
                sl = pl.ds(j * 16, 16)
                res[sl] = res[sl] + cbuf[sl]
        pltpu.sync_copy(res, deg_hbm.at[cid, pl.ds(wbase, WORDS_PT)])


def _make_seg_sum(with_deg):
    mesh = plsc.VectorSubcoreMesh(core_axis_name="c", subcore_axis_name="s",
                                  num_cores=NC, num_subcores=NS)
    out_type = [jax.ShapeDtypeStruct((NC, N_PAD, D), jnp.float32)]
    scratch = [
        pltpu.VMEM((CH,), jnp.int32),
        pltpu.VMEM((CH,), jnp.int32),
        pltpu.VMEM((CH,), jnp.int32),
        pltpu.VMEM((CH,), jnp.int32),
        pltpu.VMEM((CH, D), jnp.float32),
        pltpu.VMEM((CH, D), jnp.float32),
    ]
    if with_deg:
        out_type.append(jax.ShapeDtypeStruct((NC, N_PAD), jnp.float32))
        scratch.append(pltpu.VMEM((N_PAD,), jnp.float32))
        scratch.append(pltpu.VMEM((WORDS_PT,), jnp.float32))
        scratch.append(pltpu.VMEM((WORDS_PT,), jnp.float32))
        scratch.append(pltpu.VMEM_SHARED((N_PAD, D), jnp.float32))
        scratch.append(pltpu.VMEM_SHARED((NS, N_PAD), jnp.float32))
    else:
        scratch.append(pltpu.VMEM_SHARED((N_PAD, D), jnp.float32))
    scratch.append(pltpu.SemaphoreType.DMA)
    scratch.append(pltpu.SemaphoreType.DMA)

    return pl.kernel(
        functools.partial(_seg_sum_body, with_deg),
        out_type=tuple(out_type) if with_deg else out_type[0],
        mesh=mesh,
        scratch_types=scratch,
        compiler_params=pltpu.CompilerParams(needs_layout_passes=False),
    )


_seg_sum_deg = _make_seg_sum(True)
_seg_sum = _make_seg_sum(False)


def _prelu(v, a):
    return jnp.where(v >= 0, v, a * v)


RB = 1000


def _layer1_body(agg_ref, deg0_ref, deg1_ref, x_ref, w0l_ref, b0l_ref,
                 w0r_ref, wskip_ref, a0_ref, out_ref):
    agg = agg_ref[0] + agg_ref[1]
    deg = deg0_ref[...] + deg1_ref[...]
    mean = agg / jnp.maximum(deg, 1.0)
    xb = x_ref[...]
    a0 = a0_ref[...]
    h = (jnp.dot(mean, w0l_ref[...], preferred_element_type=jnp.float32)
         + b0l_ref[...]
         + jnp.dot(xb, w0r_ref[...], preferred_element_type=jnp.float32))
    h = _prelu(_prelu(h, a0), a0)
    out_ref[...] = h + jnp.dot(xb, wskip_ref[...],
                               preferred_element_type=jnp.float32)


def _layer2_body(agg_ref, deg0_ref, deg1_ref, h_ref, w1l_ref, b1l_ref,
                 w1r_ref, a1_ref, out_ref):
    agg = agg_ref[0] + agg_ref[1]
    deg = deg0_ref[...] + deg1_ref[...]
    mean = agg / jnp.maximum(deg, 1.0)
    h = (jnp.dot(mean, w1l_ref[...], preferred_element_type=jnp.float32)
         + b1l_ref[...]
         + jnp.dot(h_ref[...], w1r_ref[...], preferred_element_type=jnp.float32))
    out_ref[...] = _prelu(h, a1_ref[...])


_full128 = pl.BlockSpec((D, D), lambda i: (0, 0))
_row1 = pl.BlockSpec((1, D), lambda i: (0, 0))


_degspec = pl.BlockSpec((RB, 1), lambda i: (i, 0))


def _tc_layer1(aggp, deg0, deg1, x, W0l, b0l, W0r, Wskip, a0):
    return pl.pallas_call(
        _layer1_body,
        grid=(N // RB,),
        in_specs=[
            pl.BlockSpec((NC, RB, D), lambda i: (0, i, 0)),
            _degspec, _degspec,
            pl.BlockSpec((RB, D), lambda i: (i, 0)),
            _full128, _row1, _full128, _full128, _row1,
        ],
        out_specs=pl.BlockSpec((RB, D), lambda i: (i, 0)),
        out_shape=jax.ShapeDtypeStruct((N, D), jnp.float32),
    )(aggp, deg0, deg1, x, W0l, b0l, W0r, Wskip, a0)


def _tc_layer2(aggp, deg0, deg1, h, W1l, b1l, W1r, a1):
    return pl.pallas_call(
        _layer2_body,
        grid=(N // RB,),
        in_specs=[
            pl.BlockSpec((NC, RB, D), lambda i: (0, i, 0)),
            _degspec, _degspec,
            pl.BlockSpec((RB, D), lambda i: (i, 0)),
            _full128, _row1, _full128, _row1,
        ],
        out_specs=pl.BlockSpec((RB, D), lambda i: (i, 0)),
        out_shape=jax.ShapeDtypeStruct((N, D), jnp.float32),
    )(aggp, deg0, deg1, h, W1l, b1l, W1r, a1)


def kernel(x, edge_index, edge_weight, W0l, b0l, W0r, W1l, b1l, W1r,
           Wskip, a0, a1):
    del edge_weight
    src = jnp.asarray(edge_index[0], jnp.int32)
    dst = jnp.asarray(edge_index[1], jnp.int32)
    z128 = jnp.zeros((ROWS_PT, D), jnp.float32)
    b0l2 = b0l.reshape(1, D)
    b1l2 = b1l.reshape(1, D)
    a02 = a0.reshape(1, D)
    a12 = a1.reshape(1, D)

    z1d = jnp.zeros((N_PAD,), jnp.float32)
    agg1p, degp = _seg_sum_deg(x, src, dst, z128, z1d)
    deg0 = degp[0].reshape(N_PAD, 1)[:N]
    deg1 = degp[1].reshape(N_PAD, 1)[:N]
    h2in = _tc_layer1(agg1p, deg0, deg1, x, W0l, b0l2, W0r, Wskip, a02)
    agg2p = _seg_sum(h2in, src, dst, z128)
    return _tc_layer2(agg2p, deg0, deg1, h2in, W1l, b1l2, W1r, a12)


# speedup vs baseline: 11.5089x; 1.4550x over previous
import functools
import math

import jax
import jax.numpy as jnp
from jax import lax
from jax.experimental import pallas as pl
from jax.experimental.pallas import tpu as pltpu
from jax.experimental.pallas import tpu_sc as plsc

N = 10000
E = 320000
D = 128

NC = 2
NS = 16
NW = NC * NS
EPT = E // NW
CH = 80
NCHUNK = EPT // CH
N_PAD = 10240
ROWS_PT = N_PAD // NS
WORDS_PT = N_PAD // NS


def _seg_sum_body(with_deg, nb, ni, *refs):
    refs = list(refs)
    h_hbm = refs.pop(0)
    sd_hbm = refs.pop(0)
    z128_hbm = refs.pop(0)
    if with_deg:
        z1d_hbm = refs.pop(0)
    out_hbm = refs.pop(0)
    if with_deg:
        deg_hbm = refs.pop(0)
    sdv = [refs.pop(0) for _ in range(ni)]
    rows = [refs.pop(0) for _ in range(nb)]
    if with_deg:
        hist = refs.pop(0)
        cbuf = refs.pop(0)
        res = refs.pop(0)
    acc = refs.pop(0)
    if with_deg:
        stag = refs.pop(0)
    isem = [refs.pop(0) for _ in range(ni)]
    gsem = [refs.pop(0) for _ in range(nb)]
    ssem = [refs.pop(0) for _ in range(nb)]
    assert not refs

    cid = lax.axis_index("c")
    sid = lax.axis_index("s")
    wid = sid * NC + cid

    rbase = sid * ROWS_PT
    pltpu.sync_copy(z128_hbm, acc.at[pl.ds(rbase, ROWS_PT)])
    if with_deg:
        pltpu.sync_copy(z1d_hbm, hist)
    plsc.subcore_barrier()

    ones16 = jnp.ones((16,), jnp.float32)

    def idx_load(j, slot):
        pltpu.async_copy(sd_hbm.at[wid, j], sdv[slot], isem[slot])

    def idx_wait(slot):
        pltpu.make_async_copy(sd_hbm.at[wid, 0], sdv[slot],
                              isem[slot]).wait()

    def gather(b, slot):
        pltpu.async_copy(h_hbm.at[sdv[slot].at[0]], rows[b], gsem[b])

    def gather_wait(b):
        pltpu.make_async_copy(h_hbm.at[sdv[0].at[0]], rows[b],
                              gsem[b]).wait()

    def scatter(b, slot):
        pltpu.async_copy(rows[b], acc.at[sdv[slot].at[1]], ssem[b],
                         add=True)

    def scatter_wait(b):
        pltpu.make_async_copy(rows[b], acc.at[sdv[0].at[1]], ssem[b]).wait()

    def hist_upd(slot):
        if with_deg:
            for q in range(CH // 16):
                d16 = sdv[slot][1, pl.ds(q * 16, 16)]
                plsc.addupdate_scatter(hist, [d16], ones16)

    def do_chunk(j, sj, sb, j_static):
        if j_static >= nb:
            scatter_wait(sb)
        if j_static + 2 < NCHUNK:
            idx_load(j + 2, (sj + 2) % ni)
        idx_wait(sj)
        gather(sb, sj)
        if j_static >= nb - 1:
            cs = (sj - (nb - 1)) % ni
            cb = (sb - (nb - 1)) % nb
            gather_wait(cb)
            scatter(cb, cs)
            hist_upd(cs)

    idx_load(0, 0)
    idx_load(1, 1 % ni)
    for j in range(nb):
        do_chunk(j, j % ni, j % nb, j)

    unroll = math.lcm(nb, ni)
    base = nb
    n_main = ((NCHUNK - 2 - base) // unroll) * unroll
    assert n_main > 0

    def step(k, carry):
        for u in range(unroll):
            j = base + unroll * k + u
            do_chunk(j, (base + u) % ni, (base + u) % nb, base)
        return carry

    lax.fori_loop(0, n_main // unroll, step, 0)

    for j in range(base + n_main, NCHUNK):
        do_chunk(j, j % ni, j % nb, j)
    for c in range(NCHUNK - (nb - 1), NCHUNK):
        gather_wait(c % nb)
        scatter(c % nb, c % ni)
        hist_upd(c % ni)
    for b in range(nb):
        scatter_wait(b)

    if with_deg:
        pltpu.sync_copy(hist, stag.at[sid])
    plsc.subcore_barrier()

    pltpu.sync_copy(acc.at[pl.ds(rbase, ROWS_PT)],
                    out_hbm.at[cid, pl.ds(rbase, ROWS_PT)])
    if with_deg:
        wbase = sid * WORDS_PT
        pltpu.sync_copy(stag.at[0, pl.ds(wbase, WORDS_PT)], res)
        for t in range(1, NS):
            pltpu.sync_copy(stag.at[t, pl.ds(wbase, WORDS_PT)], cbuf)
            for q in range(WORDS_PT // 16):
                sl = pl.ds(q * 16, 16)
                res[sl] = res[sl] + cbuf[sl]
        pltpu.sync_copy(res, deg_hbm.at[cid, pl.ds(wbase, WORDS_PT)])


def _make_seg_sum(with_deg, nb, ni):
    mesh = plsc.VectorSubcoreMesh(core_axis_name="c", subcore_axis_name="s",
                                  num_cores=NC, num_subcores=NS)
    out_type = [jax.ShapeDtypeStruct((NC, N_PAD, D), jnp.float32)]
    if with_deg:
        out_type.append(jax.ShapeDtypeStruct((NC, N_PAD), jnp.float32))
    scratch = [pltpu.VMEM((2, CH), jnp.int32) for _ in range(ni)]
    scratch += [pltpu.VMEM((CH, D), jnp.float32) for _ in range(nb)]
    if with_deg:
        scratch.append(pltpu.VMEM((N_PAD,), jnp.float32))
        scratch.append(pltpu.VMEM((WORDS_PT,), jnp.float32))
        scratch.append(pltpu.VMEM((WORDS_PT,), jnp.float32))
    scratch.append(pltpu.VMEM_SHARED((N_PAD, D), jnp.float32))
    if with_deg:
        scratch.append(pltpu.VMEM_SHARED((NS, N_PAD), jnp.float32))
    for _ in range(ni + 2 * nb):
        scratch.append(pltpu.SemaphoreType.DMA)

    return pl.kernel(
        functools.partial(_seg_sum_body, with_deg, nb, ni),
        out_type=tuple(out_type) if with_deg else out_type[0],
        mesh=mesh,
        scratch_types=scratch,
        compiler_params=pltpu.CompilerParams(needs_layout_passes=False),
    )


_seg_sum_deg = _make_seg_sum(True, nb=2, ni=4)
_seg_sum = _make_seg_sum(False, nb=3, ni=6)


def _prelu(v, a):
    return jnp.where(v >= 0, v, a * v)


RB = 1000


def _layer1_body(agg_ref, deg0_ref, deg1_ref, x_ref, w0l_ref, b0l_ref,
                 w0r_ref, wskip_ref, a0_ref, out_ref):
    agg = agg_ref[0] + agg_ref[1]
    deg = deg0_ref[...] + deg1_ref[...]
    mean = agg / jnp.maximum(deg, 1.0)
    xb = x_ref[...]
    a0 = a0_ref[...]
    h = (jnp.dot(mean, w0l_ref[...], preferred_element_type=jnp.float32)
         + b0l_ref[...]
         + jnp.dot(xb, w0r_ref[...], preferred_element_type=jnp.float32))
    h = _prelu(_prelu(h, a0), a0)
    out_ref[...] = h + jnp.dot(xb, wskip_ref[...],
                               preferred_element_type=jnp.float32)


def _layer2_body(agg_ref, deg0_ref, deg1_ref, h_ref, w1l_ref, b1l_ref,
                 w1r_ref, a1_ref, out_ref):
    agg = agg_ref[0] + agg_ref[1]
    deg = deg0_ref[...] + deg1_ref[...]
    mean = agg / jnp.maximum(deg, 1.0)
    h = (jnp.dot(mean, w1l_ref[...], preferred_element_type=jnp.float32)
         + b1l_ref[...]
         + jnp.dot(h_ref[...], w1r_ref[...],
                   preferred_element_type=jnp.float32))
    out_ref[...] = _prelu(h, a1_ref[...])


_full128 = pl.BlockSpec((D, D), lambda i: (0, 0))
_row1 = pl.BlockSpec((1, D), lambda i: (0, 0))
_degspec = pl.BlockSpec((RB, 1), lambda i: (i, 0))


def _tc_layer1(aggp, deg0, deg1, x, W0l, b0l, W0r, Wskip, a0):
    return pl.pallas_call(
        _layer1_body,
        grid=(N // RB,),
        in_specs=[
            pl.BlockSpec((NC, RB, D), lambda i: (0, i, 0)),
            _degspec, _degspec,
            pl.BlockSpec((RB, D), lambda i: (i, 0)),
            _full128, _row1, _full128, _full128, _row1,
        ],
        out_specs=pl.BlockSpec((RB, D), lambda i: (i, 0)),
        out_shape=jax.ShapeDtypeStruct((N, D), jnp.float32),
    )(aggp, deg0, deg1, x, W0l, b0l, W0r, Wskip, a0)


def _tc_layer2(aggp, deg0, deg1, h, W1l, b1l, W1r, a1):
    return pl.pallas_call(
        _layer2_body,
        grid=(N // RB,),
        in_specs=[
            pl.BlockSpec((NC, RB, D), lambda i: (0, i, 0)),
            _degspec, _degspec,
            pl.BlockSpec((RB, D), lambda i: (i, 0)),
            _full128, _row1, _full128, _row1,
        ],
        out_specs=pl.BlockSpec((RB, D), lambda i: (i, 0)),
        out_shape=jax.ShapeDtypeStruct((N, D), jnp.float32),
    )(aggp, deg0, deg1, h, W1l, b1l, W1r, a1)


def kernel(x, edge_index, edge_weight, W0l, b0l, W0r, W1l, b1l, W1r,
           Wskip, a0, a1):
    del edge_weight
    src = jnp.asarray(edge_index[0], jnp.int32)
    dst = jnp.asarray(edge_index[1], jnp.int32)
    sd = jnp.stack([src.reshape(NW, NCHUNK, CH),
                    dst.reshape(NW, NCHUNK, CH)], axis=2)
    z128 = jnp.zeros((ROWS_PT, D), jnp.float32)
    z1d = jnp.zeros((N_PAD,), jnp.float32)
    b0l2 = b0l.reshape(1, D)
    b1l2 = b1l.reshape(1, D)
    a02 = a0.reshape(1, D)
    a12 = a1.reshape(1, D)

    agg1p, degp = _seg_sum_deg(x, sd, z128, z1d)
    deg0 = degp[0].reshape(N_PAD, 1)[:N]
    deg1 = degp[1].reshape(N_PAD, 1)[:N]
    h2in = _tc_layer1(agg1p, deg0, deg1, x, W0l, b0l2, W0r, Wskip, a02)
    agg2p = _seg_sum(h2in, sd, z128)
    return _tc_layer2(agg2p, deg0, deg1, h2in, W1l, b1l2, W1r, a12)


# speedup vs baseline: 12.1484x; 1.0556x over previous
import functools
import math

import jax
import jax.numpy as jnp
from jax import lax
from jax.experimental import pallas as pl
from jax.experimental.pallas import tpu as pltpu
from jax.experimental.pallas import tpu_sc as plsc

N = 10000
E = 320000
D = 128

NC = 2
NS = 16
NW = NC * NS
EPT = E // NW
CH = 80
NCHUNK = EPT // CH
N_PAD = 10240
ROWS_PT = N_PAD // NS
WORDS_PT = N_PAD // NS


def _seg_sum_body(with_deg, nb, ni, *refs):
    refs = list(refs)
    h_hbm = refs.pop(0)
    src_hbm = refs.pop(0)
    dst_hbm = refs.pop(0)
    z128_hbm = refs.pop(0)
    if with_deg:
        z1d_hbm = refs.pop(0)
    out_hbm = refs.pop(0)
    if with_deg:
        deg_hbm = refs.pop(0)
    srcv = [refs.pop(0) for _ in range(ni)]
    dstv = [refs.pop(0) for _ in range(ni)]
    rows = [refs.pop(0) for _ in range(nb)]
    if with_deg:
        hist = refs.pop(0)
        cbuf = refs.pop(0)
        res = refs.pop(0)
    acc = refs.pop(0)
    if with_deg:
        stag = refs.pop(0)
    isem = [refs.pop(0) for _ in range(ni)]
    idsem = [refs.pop(0) for _ in range(ni)]
    gsem = [refs.pop(0) for _ in range(nb)]
    ssem = [refs.pop(0) for _ in range(nb)]
    assert not refs

    cid = lax.axis_index("c")
    sid = lax.axis_index("s")
    wid = sid * NC + cid

    rbase = sid * ROWS_PT
    pltpu.sync_copy(z128_hbm, acc.at[pl.ds(rbase, ROWS_PT)])
    if with_deg:
        pltpu.sync_copy(z1d_hbm, hist)
    plsc.subcore_barrier()

    ones16 = jnp.ones((16,), jnp.float32)

    ebase = wid * EPT

    def idx_load(j, slot):
        off = ebase + j * CH
        pltpu.async_copy(src_hbm.at[pl.ds(off, CH)], srcv[slot], isem[slot])
        pltpu.async_copy(dst_hbm.at[pl.ds(off, CH)], dstv[slot], idsem[slot])

    def idx_wait(slot):
        pltpu.make_async_copy(src_hbm.at[pl.ds(0, CH)], srcv[slot],
                              isem[slot]).wait()
        pltpu.make_async_copy(dst_hbm.at[pl.ds(0, CH)], dstv[slot],
                              idsem[slot]).wait()

    def gather(b, slot):
        pltpu.async_copy(h_hbm.at[srcv[slot]], rows[b], gsem[b])

    def gather_wait(b):
        pltpu.make_async_copy(h_hbm.at[srcv[0]], rows[b], gsem[b]).wait()

    def scatter(b, slot):
        pltpu.async_copy(rows[b], acc.at[dstv[slot]], ssem[b], add=True)

    def scatter_wait(b):
        pltpu.make_async_copy(rows[b], acc.at[dstv[0]], ssem[b]).wait()

    def hist_upd(slot):
        if with_deg:
            for q in range(CH // 16):
                d16 = dstv[slot][pl.ds(q * 16, 16)]
                plsc.addupdate_scatter(hist, [d16], ones16)

    def do_chunk(j, sj, sb, j_static):
        if j_static >= nb:
            scatter_wait(sb)
        if j_static + 2 < NCHUNK:
            idx_load(j + 2, (sj + 2) % ni)
        idx_wait(sj)
        gather(sb, sj)
        if j_static >= nb - 1:
            cs = (sj - (nb - 1)) % ni
            cb = (sb - (nb - 1)) % nb
            gather_wait(cb)
            scatter(cb, cs)
            hist_upd(cs)

    idx_load(0, 0)
    idx_load(1, 1 % ni)
    for j in range(nb):
        do_chunk(j, j % ni, j % nb, j)

    unroll = math.lcm(nb, ni)
    base = nb
    n_main = ((NCHUNK - 2 - base) // unroll) * unroll
    assert n_main > 0

    def step(k, carry):
        for u in range(unroll):
            j = base + unroll * k + u
            do_chunk(j, (base + u) % ni, (base + u) % nb, base)
        return carry

    lax.fori_loop(0, n_main // unroll, step, 0)

    for j in range(base + n_main, NCHUNK):
        do_chunk(j, j % ni, j % nb, j)
    for c in range(NCHUNK - (nb - 1), NCHUNK):
        gather_wait(c % nb)
        scatter(c % nb, c % ni)
        hist_upd(c % ni)
    for b in range(nb):
        scatter_wait(b)

    if with_deg:
        pltpu.sync_copy(hist, stag.at[sid])
    plsc.subcore_barrier()

    pltpu.sync_copy(acc.at[pl.ds(rbase, ROWS_PT)],
                    out_hbm.at[cid, pl.ds(rbase, ROWS_PT)])
    if with_deg:
        wbase = sid * WORDS_PT
        pltpu.sync_copy(stag.at[0, pl.ds(wbase, WORDS_PT)], res)
        for t in range(1, NS):
            pltpu.sync_copy(stag.at[t, pl.ds(wbase, WORDS_PT)], cbuf)
            for q in range(WORDS_PT // 16):
                sl = pl.ds(q * 16, 16)
                res[sl] = res[sl] + cbuf[sl]
        pltpu.sync_copy(res, deg_hbm.at[cid, pl.ds(wbase, WORDS_PT)])


def _make_seg_sum(with_deg, nb, ni):
    mesh = plsc.VectorSubcoreMesh(core_axis_name="c", subcore_axis_name="s",
                                  num_cores=NC, num_subcores=NS)
    out_type = [jax.ShapeDtypeStruct((NC, N_PAD, D), jnp.float32)]
    if with_deg:
        out_type.append(jax.ShapeDtypeStruct((NC, N_PAD), jnp.float32))
    scratch = [pltpu.VMEM((CH,), jnp.int32) for _ in range(2 * ni)]
    scratch += [pltpu.VMEM((CH, D), jnp.float32) for _ in range(nb)]
    if with_deg:
        scratch.append(pltpu.VMEM((N_PAD,), jnp.float32))
        scratch.append(pltpu.VMEM((WORDS_PT,), jnp.float32))
        scratch.append(pltpu.VMEM((WORDS_PT,), jnp.float32))
    scratch.append(pltpu.VMEM_SHARED((N_PAD, D), jnp.float32))
    if with_deg:
        scratch.append(pltpu.VMEM_SHARED((NS, N_PAD), jnp.float32))
    for _ in range(2 * ni + 2 * nb):
        scratch.append(pltpu.SemaphoreType.DMA)

    return pl.kernel(
        functools.partial(_seg_sum_body, with_deg, nb, ni),
        out_type=tuple(out_type) if with_deg else out_type[0],
        mesh=mesh,
        scratch_types=scratch,
        compiler_params=pltpu.CompilerParams(needs_layout_passes=False),
    )


_seg_sum_deg = _make_seg_sum(True, nb=2, ni=4)
_seg_sum = _make_seg_sum(False, nb=3, ni=6)


def _prelu(v, a):
    return jnp.where(v >= 0, v, a * v)


RB = 1000


def _layer1_body(agg_ref, deg0_ref, deg1_ref, x_ref, w0l_ref, b0l_ref,
                 w0r_ref, wskip_ref, a0_ref, out_ref):
    agg = agg_ref[0] + agg_ref[1]
    deg = deg0_ref[...] + deg1_ref[...]
    mean = agg / jnp.maximum(deg, 1.0)
    xb = x_ref[...]
    a0 = a0_ref[...]
    h = (jnp.dot(mean, w0l_ref[...], preferred_element_type=jnp.float32)
         + b0l_ref[...]
         + jnp.dot(xb, w0r_ref[...], preferred_element_type=jnp.float32))
    h = _prelu(_prelu(h, a0), a0)
    out_ref[...] = h + jnp.dot(xb, wskip_ref[...],
                               preferred_element_type=jnp.float32)


def _layer2_body(agg_ref, deg0_ref, deg1_ref, h_ref, w1l_ref, b1l_ref,
                 w1r_ref, a1_ref, out_ref):
    agg = agg_ref[0] + agg_ref[1]
    deg = deg0_ref[...] + deg1_ref[...]
    mean = agg / jnp.maximum(deg, 1.0)
    h = (jnp.dot(mean, w1l_ref[...], preferred_element_type=jnp.float32)
         + b1l_ref[...]
         + jnp.dot(h_ref[...], w1r_ref[...],
                   preferred_element_type=jnp.float32))
    out_ref[...] = _prelu(h, a1_ref[...])


_full128 = pl.BlockSpec((D, D), lambda i: (0, 0))
_row1 = pl.BlockSpec((1, D), lambda i: (0, 0))
_degspec = pl.BlockSpec((RB, 1), lambda i: (i, 0))


def _tc_layer1(aggp, deg0, deg1, x, W0l, b0l, W0r, Wskip, a0):
    return pl.pallas_call(
        _layer1_body,
        grid=(N // RB,),
        in_specs=[
            pl.BlockSpec((NC, RB, D), lambda i: (0, i, 0)),
            _degspec, _degspec,
            pl.BlockSpec((RB, D), lambda i: (i, 0)),
            _full128, _row1, _full128, _full128, _row1,
        ],
        out_specs=pl.BlockSpec((RB, D), lambda i: (i, 0)),
        out_shape=jax.ShapeDtypeStruct((N, D), jnp.float32),
    )(aggp, deg0, deg1, x, W0l, b0l, W0r, Wskip, a0)


def _tc_layer2(aggp, deg0, deg1, h, W1l, b1l, W1r, a1):
    return pl.pallas_call(
        _layer2_body,
        grid=(N // RB,),
        in_specs=[
            pl.BlockSpec((NC, RB, D), lambda i: (0, i, 0)),
            _degspec, _degspec,
            pl.BlockSpec((RB, D), lambda i: (i, 0)),
            _full128, _row1, _full128, _row1,
        ],
        out_specs=pl.BlockSpec((RB, D), lambda i: (i, 0)),
        out_shape=jax.ShapeDtypeStruct((N, D), jnp.float32),
    )(aggp, deg0, deg1, h, W1l, b1l, W1r, a1)


def kernel(x, edge_index, edge_weight, W0l, b0l, W0r, W1l, b1l, W1r,
           Wskip, a0, a1):
    del edge_weight
    src = jnp.asarray(edge_index[0], jnp.int32)
    dst = jnp.asarray(edge_index[1], jnp.int32)
    z128 = jnp.zeros((ROWS_PT, D), jnp.float32)
    z1d = jnp.zeros((N_PAD,), jnp.float32)
    b0l2 = b0l.reshape(1, D)
    b1l2 = b1l.reshape(1, D)
    a02 = a0.reshape(1, D)
    a12 = a1.reshape(1, D)

    agg1p, degp = _seg_sum_deg(x, src, dst, z128, z1d)
    deg0 = degp[0].reshape(N_PAD, 1)[:N]
    deg1 = degp[1].reshape(N_PAD, 1)[:N]
    h2in = _tc_layer1(agg1p, deg0, deg1, x, W0l, b0l2, W0r, Wskip, a02)
    agg2p = _seg_sum(h2in, src, dst, z128)
    return _tc_layer2(agg2p, deg0, deg1, h2in, W1l, b1l2, W1r, a12)


# speedup vs baseline: 12.2795x; 1.0108x over previous
import functools
import math

import jax
import jax.numpy as jnp
from jax import lax
from jax.experimental import pallas as pl
from jax.experimental.pallas import tpu as pltpu
from jax.experimental.pallas import tpu_sc as plsc

N = 10000
E = 320000
D = 128

NC = 2
NS = 16
NW = NC * NS
EPT = E // NW
CH = 80
NCHUNK = EPT // CH
N_PAD = 10240
ROWS_PT = N_PAD // NS
WORDS_PT = N_PAD // NS


def _seg_sum_body(with_deg, nb, ni, *refs):
    refs = list(refs)
    h_hbm = refs.pop(0)
    src_hbm = refs.pop(0)
    dst_hbm = refs.pop(0)
    z128_hbm = refs.pop(0)
    if with_deg:
        z1d_hbm = refs.pop(0)
    out_hbm = refs.pop(0)
    if with_deg:
        deg_hbm = refs.pop(0)
    srcv = [refs.pop(0) for _ in range(ni)]
    dstv = [refs.pop(0) for _ in range(ni)]
    rows = [refs.pop(0) for _ in range(nb)]
    if with_deg:
        hist = refs.pop(0)
        cbuf = refs.pop(0)
        res = refs.pop(0)
    acc = refs.pop(0)
    if with_deg:
        stag = refs.pop(0)
    isem = [refs.pop(0) for _ in range(ni)]
    idsem = [refs.pop(0) for _ in range(ni)]
    gsem = [refs.pop(0) for _ in range(nb)]
    ssem = [refs.pop(0) for _ in range(nb)]
    assert not refs

    cid = lax.axis_index("c")
    sid = lax.axis_index("s")
    wid = sid * NC + cid

    rbase = sid * ROWS_PT
    pltpu.sync_copy(z128_hbm, acc.at[pl.ds(rbase, ROWS_PT)])
    if with_deg:
        pltpu.sync_copy(z1d_hbm, hist)
    plsc.subcore_barrier()

    ones16 = jnp.ones((16,), jnp.float32)

    ebase = wid * EPT

    def idx_load(j, slot):
        off = ebase + j * CH
        pltpu.async_copy(src_hbm.at[pl.ds(off, CH)], srcv[slot], isem[slot])
        pltpu.async_copy(dst_hbm.at[pl.ds(off, CH)], dstv[slot], idsem[slot])

    def idx_wait(slot):
        pltpu.make_async_copy(src_hbm.at[pl.ds(0, CH)], srcv[slot],
                              isem[slot]).wait()
        pltpu.make_async_copy(dst_hbm.at[pl.ds(0, CH)], dstv[slot],
                              idsem[slot]).wait()

    def gather(b, slot):
        pltpu.async_copy(h_hbm.at[srcv[slot]], rows[b], gsem[b])

    def gather_wait(b):
        pltpu.make_async_copy(h_hbm.at[srcv[0]], rows[b], gsem[b]).wait()

    def scatter(b, slot):
        pltpu.async_copy(rows[b], acc.at[dstv[slot]], ssem[b], add=True)

    def scatter_wait(b):
        pltpu.make_async_copy(rows[b], acc.at[dstv[0]], ssem[b]).wait()

    def hist_upd(slot):
        if with_deg:
            for q in range(CH // 16):
                d16 = dstv[slot][pl.ds(q * 16, 16)]
                plsc.addupdate_scatter(hist, [d16], ones16)

    def do_chunk(j, sj, sb, j_static):
        if j_static >= nb:
            scatter_wait(sb)
        if j_static + 2 < NCHUNK:
            idx_load(j + 2, (sj + 2) % ni)
        idx_wait(sj)
        gather(sb, sj)
        if j_static >= nb - 1:
            cs = (sj - (nb - 1)) % ni
            cb = (sb - (nb - 1)) % nb
            gather_wait(cb)
            scatter(cb, cs)
            hist_upd(cs)

    idx_load(0, 0)
    idx_load(1, 1 % ni)
    for j in range(nb):
        do_chunk(j, j % ni, j % nb, j)

    unroll = math.lcm(nb, ni)
    base = nb
    n_main = ((NCHUNK - 2 - base) // unroll) * unroll
    assert n_main > 0

    def step(k, carry):
        for u in range(unroll):
            j = base + unroll * k + u
            do_chunk(j, (base + u) % ni, (base + u) % nb, base)
        return carry

    lax.fori_loop(0, n_main // unroll, step, 0)

    for j in range(base + n_main, NCHUNK):
        do_chunk(j, j % ni, j % nb, j)
    for c in range(NCHUNK - (nb - 1), NCHUNK):
        gather_wait(c % nb)
        scatter(c % nb, c % ni)
        hist_upd(c % ni)
    for b in range(nb):
        scatter_wait(b)

    if with_deg:
        pltpu.sync_copy(hist, stag.at[sid])
    plsc.subcore_barrier()

    pltpu.sync_copy(acc.at[pl.ds(rbase, ROWS_PT)],
                    out_hbm.at[cid, pl.ds(rbase, ROWS_PT)])
    if with_deg:
        wbase = sid * WORDS_PT
        pltpu.sync_copy(stag.at[0, pl.ds(wbase, WORDS_PT)], res)
        for t in range(1, NS):
            pltpu.sync_copy(stag.at[t, pl.ds(wbase, WORDS_PT)], cbuf)
            for q in range(WORDS_PT // 16):
                sl = pl.ds(q * 16, 16)
                res[sl] = res[sl] + cbuf[sl]
        pltpu.sync_copy(res, deg_hbm.at[cid, pl.ds(wbase, WORDS_PT)])


def _make_seg_sum(with_deg, nb, ni):
    mesh = plsc.VectorSubcoreMesh(core_axis_name="c", subcore_axis_name="s",
                                  num_cores=NC, num_subcores=NS)
    out_type = [jax.ShapeDtypeStruct((NC, N_PAD, D), jnp.float32)]
    if with_deg:
        out_type.append(jax.ShapeDtypeStruct((NC, N_PAD), jnp.float32))
    scratch = [pltpu.VMEM((CH,), jnp.int32) for _ in range(2 * ni)]
    scratch += [pltpu.VMEM((CH, D), jnp.float32) for _ in range(nb)]
    if with_deg:
        scratch.append(pltpu.VMEM((N_PAD,), jnp.float32))
        scratch.append(pltpu.VMEM((WORDS_PT,), jnp.float32))
        scratch.append(pltpu.VMEM((WORDS_PT,), jnp.float32))
    scratch.append(pltpu.VMEM_SHARED((N_PAD, D), jnp.float32))
    if with_deg:
        scratch.append(pltpu.VMEM_SHARED((NS, N_PAD), jnp.float32))
    for _ in range(2 * ni + 2 * nb):
        scratch.append(pltpu.SemaphoreType.DMA)

    return pl.kernel(
        functools.partial(_seg_sum_body, with_deg, nb, ni),
        out_type=tuple(out_type) if with_deg else out_type[0],
        mesh=mesh,
        scratch_types=scratch,
        compiler_params=pltpu.CompilerParams(needs_layout_passes=False),
    )


_seg_sum_deg = _make_seg_sum(True, nb=2, ni=4)
_seg_sum = _make_seg_sum(False, nb=4, ni=6)


def _prelu(v, a):
    return jnp.where(v >= 0, v, a * v)


RB = 1000


def _layer1_body(agg_ref, deg0_ref, deg1_ref, x_ref, w0l_ref, b0l_ref,
                 w0r_ref, wskip_ref, a0_ref, out_ref):
    agg = agg_ref[0] + agg_ref[1]
    deg = deg0_ref[...] + deg1_ref[...]
    mean = agg / jnp.maximum(deg, 1.0)
    xb = x_ref[...]
    a0 = a0_ref[...]
    h = (jnp.dot(mean, w0l_ref[...], preferred_element_type=jnp.float32)
         + b0l_ref[...]
         + jnp.dot(xb, w0r_ref[...], preferred_element_type=jnp.float32))
    h = _prelu(_prelu(h, a0), a0)
    out_ref[...] = h + jnp.dot(xb, wskip_ref[...],
                               preferred_element_type=jnp.float32)


def _layer2_body(agg_ref, deg0_ref, deg1_ref, h_ref, w1l_ref, b1l_ref,
                 w1r_ref, a1_ref, out_ref):
    agg = agg_ref[0] + agg_ref[1]
    deg = deg0_ref[...] + deg1_ref[...]
    mean = agg / jnp.maximum(deg, 1.0)
    h = (jnp.dot(mean, w1l_ref[...], preferred_element_type=jnp.float32)
         + b1l_ref[...]
         + jnp.dot(h_ref[...], w1r_ref[...],
                   preferred_element_type=jnp.float32))
    out_ref[...] = _prelu(h, a1_ref[...])


_full128 = pl.BlockSpec((D, D), lambda i: (0, 0))
_row1 = pl.BlockSpec((1, D), lambda i: (0, 0))
_degspec = pl.BlockSpec((RB, 1), lambda i: (i, 0))


def _tc_layer1(aggp, deg0, deg1, x, W0l, b0l, W0r, Wskip, a0):
    return pl.pallas_call(
        _layer1_body,
        grid=(N // RB,),
        in_specs=[
            pl.BlockSpec((NC, RB, D), lambda i: (0, i, 0)),
            _degspec, _degspec,
            pl.BlockSpec((RB, D), lambda i: (i, 0)),
            _full128, _row1, _full128, _full128, _row1,
        ],
        out_specs=pl.BlockSpec((RB, D), lambda i: (i, 0)),
        out_shape=jax.ShapeDtypeStruct((N, D), jnp.float32),
    )(aggp, deg0, deg1, x, W0l, b0l, W0r, Wskip, a0)


def _tc_layer2(aggp, deg0, deg1, h, W1l, b1l, W1r, a1):
    return pl.pallas_call(
        _layer2_body,
        grid=(N // RB,),
        in_specs=[
            pl.BlockSpec((NC, RB, D), lambda i: (0, i, 0)),
            _degspec, _degspec,
            pl.BlockSpec((RB, D), lambda i: (i, 0)),
            _full128, _row1, _full128, _row1,
        ],
        out_specs=pl.BlockSpec((RB, D), lambda i: (i, 0)),
        out_shape=jax.ShapeDtypeStruct((N, D), jnp.float32),
    )(aggp, deg0, deg1, h, W1l, b1l, W1r, a1)


def kernel(x, edge_index, edge_weight, W0l, b0l, W0r, W1l, b1l, W1r,
           Wskip, a0, a1):
    del edge_weight
    src = jnp.asarray(edge_index[0], jnp.int32)
    dst = jnp.asarray(edge_index[1], jnp.int32)
    z128 = jnp.zeros((ROWS_PT, D), jnp.float32)
    z1d = jnp.zeros((N_PAD,), jnp.float32)
    b0l2 = b0l.reshape(1, D)
    b1l2 = b1l.reshape(1, D)
    a02 = a0.reshape(1, D)
    a12 = a1.reshape(1, D)

    agg1p, degp = _seg_sum_deg(x, src, dst, z128, z1d)
    deg0 = degp[0].reshape(N_PAD, 1)[:N]
    deg1 = degp[1].reshape(N_PAD, 1)[:N]
    h2in = _tc_layer1(agg1p, deg0, deg1, x, W0l, b0l2, W0r, Wskip, a02)
    agg2p = _seg_sum(h2in, src, dst, z128)
    return _tc_layer2(agg2p, deg0, deg1, h2in, W1l, b1l2, W1r, a12)
